# baseline (speedup 1.0000x reference)
import jax
import jax.numpy as jnp
from jax import lax
from jax.experimental import pallas as pl
from jax.experimental.pallas import tpu as pltpu
from jax.experimental.pallas import tpu_sc as plsc

N = 10000
E = 320000
D = 128
DE = 16

NC = 2
NS = 16
NW = NC * NS
EPW = E // NW
EPWP = 10240
B1 = 128
NB1 = EPWP // B1
B2 = 80
NB2 = EPWP // B2
RPT = N // NS
LRELU = 0.2
PK = 16384


def _proj_body(x_ref, wl_ref, bl_ref, wr_ref, br_ref, ea_ref, we_ref,
               xl_ref, xr_ref, ew_ref, easum_ref):
    xb = x_ref[...]
    xl_ref[...] = jnp.dot(xb, wl_ref[...], preferred_element_type=jnp.float32) + bl_ref[...]
    xr_ref[...] = jnp.dot(xb, wr_ref[...], preferred_element_type=jnp.float32) + br_ref[...]
    ea = ea_ref[...]
    ew_ref[...] = jnp.dot(ea, we_ref[...], preferred_element_type=jnp.float32)
    part = jnp.sum(ea, axis=0, keepdims=True)

    @pl.when(pl.program_id(0) == 0)
    def _():
        easum_ref[...] = part

    @pl.when(pl.program_id(0) != 0)
    def _():
        easum_ref[...] = easum_ref[...] + part


def _proj(x, W_l, b_l, W_r, b_r, edge_attr, W_e):
    grid = (25,)
    nblk = N // 25
    eblk = E // 25
    return pl.pallas_call(
        _proj_body,
        grid=grid,
        in_specs=[
            pl.BlockSpec((nblk, D), lambda i: (i, 0)),
            pl.BlockSpec((D, D), lambda i: (0, 0)),
            pl.BlockSpec((1, D), lambda i: (0, 0)),
            pl.BlockSpec((D, D), lambda i: (0, 0)),
            pl.BlockSpec((1, D), lambda i: (0, 0)),
            pl.BlockSpec((eblk, DE), lambda i: (i, 0)),
            pl.BlockSpec((DE, D), lambda i: (0, 0)),
        ],
        out_specs=[
            pl.BlockSpec((nblk, D), lambda i: (i, 0)),
            pl.BlockSpec((nblk, D), lambda i: (i, 0)),
            pl.BlockSpec((eblk, D), lambda i: (i, 0)),
            pl.BlockSpec((1, DE), lambda i: (0, 0)),
        ],
        out_shape=[
            jax.ShapeDtypeStruct((N, D), jnp.float32),
            jax.ShapeDtypeStruct((N, D), jnp.float32),
            jax.ShapeDtypeStruct((E, D), jnp.float32),
            jax.ShapeDtypeStruct((1, DE), jnp.float32),
        ],
    )(x, W_l, b_l.reshape(1, D), W_r, b_r.reshape(1, D), edge_attr, W_e)


def _sc_alpha_body(xl_hbm, xr_hbm, ew_hbm, sd_hbm, att_hbm,
                   alpha_hbm, maxes_hbm,
                   sd_v, srcb_v, dstb_v, xl_v, xr_v, ew_v, att_v, alpha_v,
                   stage_v, sem):
    cid = lax.axis_index("c")
    sid = lax.axis_index("s")
    wid = cid * NS + sid

    pltpu.sync_copy(sd_hbm.at[wid], sd_v)
    pltpu.sync_copy(att_hbm, att_v)

    lane_iota = lax.iota(jnp.int32, 16)

    def unpack(b, p):
        for g0 in range(B1 // 16):
            sl = pl.ds(g0 * 16, 16)
            packed = sd_v[b, sl]
            srcb_v[p, sl] = lax.shift_right_logical(packed, 14)
            dstb_v[p, sl] = lax.bitwise_and(packed, PK - 1)

    def issue(b, p):
        pltpu.async_copy(xl_hbm.at[srcb_v.at[p]], xl_v.at[p], sem)
        pltpu.async_copy(xr_hbm.at[dstb_v.at[p]], xr_v.at[p], sem)
        base = jnp.minimum(wid * EPW + b * B1, E - B1)
        pltpu.async_copy(ew_hbm.at[pl.ds(base, B1)], ew_v.at[p], sem)

    def drain(p):
        pltpu.make_async_copy(xl_hbm.at[pl.ds(0, B1)], xl_v.at[p], sem).wait()
        pltpu.make_async_copy(xl_hbm.at[pl.ds(0, B1)], xr_v.at[p], sem).wait()
        pltpu.make_async_copy(ew_hbm.at[pl.ds(0, B1)], ew_v.at[p], sem).wait()

    unpack(0, 0)
    issue(0, 0)

    def batch_body(b, _):
        p = lax.rem(b, 2)

        @pl.when(b + 1 < NB1)
        def _():
            unpack(b + 1, 1 - p)

        drain(p)

        @pl.when(b + 1 < NB1)
        def _():
            issue(b + 1, 1 - p)

        @plsc.parallel_loop(0, B1 // 16, unroll=2)
        def group_body(g):
            alpha16 = jnp.zeros((16,), jnp.float32)
            for e in range(16):
                r = g * 16 + e
                acc0 = jnp.zeros((16,), jnp.float32)
                acc1 = jnp.zeros((16,), jnp.float32)
                for v in range(D // 16):
                    sl = pl.ds(v * 16, 16)
                    m = xl_v[p, r, sl] + xr_v[p, r, sl] + ew_v[p, r, sl]
                    m = jnp.maximum(m, LRELU * m)
                    if v % 2 == 0:
                        acc0 = acc0 + m * att_v[sl]
                    else:
                        acc1 = acc1 + m * att_v[sl]
                s = jnp.sum(acc0 + acc1)
                alpha16 = jnp.where(lane_iota == e, s, alpha16)
            eidx = b * B1 + g * 16 + lane_iota
            alpha16 = jnp.where(eidx < EPW, alpha16, -3.0e38)
            alpha_v[pl.ds(b * B1 + g * 16, 16)] = alpha16

        return 0

    lax.fori_loop(0, NB1, batch_body, 0)

    def max_body(i, m16):
        return jnp.maximum(m16, alpha_v[pl.ds(i * 16, 16)])

    m16 = lax.fori_loop(0, EPWP // 16, max_body,
                        jnp.full((16,), -3.0e38, jnp.float32))
    pltpu.sync_copy(alpha_v, alpha_hbm.at[wid])
    stage_v[...] = jnp.full((16,), jnp.max(m16), jnp.float32)
    pltpu.sync_copy(stage_v, maxes_hbm.at[wid])


def _sc_alpha(x_l, x_r, ew, sd1, att):
    mesh = plsc.VectorSubcoreMesh(core_axis_name="c", subcore_axis_name="s")
    f = pl.kernel(
        _sc_alpha_body,
        out_type=[
            jax.ShapeDtypeStruct((NW, EPWP), jnp.float32),
            jax.ShapeDtypeStruct((NW, 16), jnp.float32),
        ],
        mesh=mesh,
        compiler_params=pltpu.CompilerParams(needs_layout_passes=False,
                                             use_tc_tiling_on_sc=False),
        scratch_types=[
            pltpu.VMEM((NB1, B1), jnp.int32),
            pltpu.VMEM((2, B1), jnp.int32),
            pltpu.VMEM((2, B1), jnp.int32),
            pltpu.VMEM((2, B1, D), jnp.float32),
            pltpu.VMEM((2, B1, D), jnp.float32),
            pltpu.VMEM((2, B1, D), jnp.float32),
            pltpu.VMEM((D,), jnp.float32),
            pltpu.VMEM((EPWP,), jnp.float32),
            pltpu.VMEM((16,), jnp.float32),
            pltpu.SemaphoreType.DMA,
        ],
    )
    return f(x_l, x_r, ew, sd1, att)


def _sc_agg_body(xl_hbm, sd_hbm, alpha_hbm, maxes_hbm,
                 acc_hbm, asum_hbm,
                 sd_v, srcb_v, dstb_v, rows_v, aexp_v, alpha_b, maxes_v,
                 acc_sh, asum_sh, sem):
    cid = lax.axis_index("c")
    sid = lax.axis_index("s")
    wid = cid * NS + sid

    def zbuf_body(i, _):
        for v in range(D // 16):
            rows_v[0, i, pl.ds(v * 16, 16)] = jnp.zeros((16,), jnp.float32)
        aexp_v[i, :] = jnp.zeros((16,), jnp.float32)
        return 0

    lax.fori_loop(0, B2, zbuf_body, 0)

    nchunk = N // 16

    def zspm_body(i, _):
        c = sid + NS * i

        @pl.when(c < nchunk)
        def _():
            pltpu.sync_copy(rows_v.at[0, pl.ds(0, 16)],
                            acc_sh.at[pl.ds(c * 16, 16)])
            pltpu.sync_copy(aexp_v.at[pl.ds(0, 16)],
                            asum_sh.at[pl.ds(c * 16, 16)])
        return 0

    lax.fori_loop(0, (nchunk + NS - 1) // NS, zspm_body, 0)
    plsc.subcore_barrier()

    pltpu.sync_copy(maxes_hbm, maxes_v)
    gv = maxes_v[0, :]
    for w in range(1, NW):
        gv = jnp.maximum(gv, maxes_v[w, :])
    gmax = jnp.max(gv)

    pltpu.sync_copy(sd_hbm.at[wid], sd_v)

    lane0 = (lax.iota(jnp.int32, 16) == 0).astype(jnp.float32)

    def unpack(b, p):
        for g0 in range(B2 // 16):
            sl = pl.ds(g0 * 16, 16)
            packed = sd_v[b, sl]
            srcb_v[p, sl] = lax.shift_right_logical(packed, 14)
            dstb_v[p, sl] = lax.bitwise_and(packed, PK - 1)

    def issue(b, p):
        pltpu.async_copy(xl_hbm.at[srcb_v.at[p]], rows_v.at[p], sem)
        pltpu.async_copy(alpha_hbm.at[wid, pl.ds(b * B2, B2)], alpha_b.at[p],
                         sem)

    def drain(p):
        pltpu.make_async_copy(xl_hbm.at[pl.ds(0, B2)], rows_v.at[p], sem).wait()
        pltpu.make_async_copy(alpha_hbm.at[0, pl.ds(0, B2)], alpha_b.at[p],
                              sem).wait()

    unpack(0, 0)
    issue(0, 0)

    def batch_body(b, _):
        p = lax.rem(b, 2)

        @pl.when(b + 1 < NB2)
        def _():
            unpack(b + 1, 1 - p)

        drain(p)

        @pl.when(b + 1 < NB2)
        def _():
            issue(b + 1, 1 - p)

        @plsc.parallel_loop(0, B2 // 16, unroll=2)
        def group_body(g):
            av = alpha_b[p, pl.ds(g * 16, 16)]
            ev16 = jnp.exp(av - gmax)
            for e in range(16):
                r = g * 16 + e
                evb = ev16.at[jnp.full((16,), e, jnp.int32)].get(
                    mode="promise_in_bounds")
                for v in range(D // 16):
                    sl = pl.ds(v * 16, 16)
                    rows_v[p, r, sl] = rows_v[p, r, sl] * evb
                aexp_v[r, :] = evb * lane0

        pltpu.sync_copy(rows_v.at[p], acc_sh.at[dstb_v.at[p]], add=True)
        pltpu.sync_copy(aexp_v, asum_sh.at[dstb_v.at[p]], add=True)
        return 0

    lax.fori_loop(0, NB2, batch_body, 0)
    plsc.subcore_barrier()

    pltpu.sync_copy(acc_sh.at[pl.ds(sid * RPT, RPT)],
                    acc_hbm.at[cid, pl.ds(sid * RPT, RPT)])
    pltpu.sync_copy(asum_sh.at[pl.ds(sid * RPT, RPT)],
                    asum_hbm.at[cid, pl.ds(sid * RPT, RPT)])


def _sc_agg(x_l, sd, alpha, maxes):
    mesh = plsc.VectorSubcoreMesh(core_axis_name="c", subcore_axis_name="s")
    f = pl.kernel(
        _sc_agg_body,
        out_type=[
            jax.ShapeDtypeStruct((NC, N, D), jnp.float32),
            jax.ShapeDtypeStruct((NC, N, 16), jnp.float32),
        ],
        mesh=mesh,
        compiler_params=pltpu.CompilerParams(needs_layout_passes=False,
                                             use_tc_tiling_on_sc=False),
        scratch_types=[
            pltpu.VMEM((NB2, B2), jnp.int32),
            pltpu.VMEM((2, B2), jnp.int32),
            pltpu.VMEM((2, B2), jnp.int32),
            pltpu.VMEM((2, B2, D), jnp.float32),
            pltpu.VMEM((B2, 16), jnp.float32),
            pltpu.VMEM((2, B2), jnp.float32),
            pltpu.VMEM((NW, 16), jnp.float32),
            pltpu.VMEM_SHARED((N, D), jnp.float32),
            pltpu.VMEM_SHARED((N, 16), jnp.float32),
            pltpu.SemaphoreType.DMA,
        ],
    )
    return f(x_l, sd, alpha, maxes)


def _final_body(acc0_ref, acc1_ref, as0_ref, as1_ref, xl_ref, xr_ref,
                easum_ref, we_ref, att_ref, bias_ref, maxes_ref, out_ref):
    gmax = jnp.max(maxes_ref[...])
    ea_mean = easum_ref[...] * (1.0 / E)
    c = jnp.sum(we_ref[...] * ea_mean.reshape(DE, 1), axis=0, keepdims=True)
    xl = xl_ref[...]
    m_self = xl + xr_ref[...] + c
    m_self = jnp.maximum(m_self, LRELU * m_self)
    alpha_self = jnp.sum(m_self * att_ref[...], axis=1, keepdims=True)
    self_aexp = jnp.exp(alpha_self - gmax)
    asum = (jnp.sum(as0_ref[...], axis=1, keepdims=True)
            + jnp.sum(as1_ref[...], axis=1, keepdims=True) + self_aexp)
    numer = acc0_ref[...] + acc1_ref[...] + self_aexp * xl
    h = numer / (asum + 1e-16) + bias_ref[...]
    out_ref[...] = jnp.maximum(h, 0.01 * h)


def _final(acc0, acc1, asum0, asum1, x_l, x_r, ea_sum, W_e, att, bias, maxes):
    blk = 1000
    grid = (N // blk,)
    return pl.pallas_call(
        _final_body,
        grid=grid,
        in_specs=[
            pl.BlockSpec((blk, D), lambda i: (i, 0)),
            pl.BlockSpec((blk, D), lambda i: (i, 0)),
            pl.BlockSpec((blk, 16), lambda i: (i, 0)),
            pl.BlockSpec((blk, 16), lambda i: (i, 0)),
            pl.BlockSpec((blk, D), lambda i: (i, 0)),
            pl.BlockSpec((blk, D), lambda i: (i, 0)),
            pl.BlockSpec((1, DE), lambda i: (0, 0)),
            pl.BlockSpec((DE, D), lambda i: (0, 0)),
            pl.BlockSpec((1, D), lambda i: (0, 0)),
            pl.BlockSpec((1, D), lambda i: (0, 0)),
            pl.BlockSpec((NW, 16), lambda i: (0, 0)),
        ],
        out_specs=pl.BlockSpec((blk, D), lambda i: (i, 0)),
        out_shape=jax.ShapeDtypeStruct((N, D), jnp.float32),
    )(acc0, acc1, asum0, asum1, x_l, x_r, ea_sum, W_e,
      att.reshape(1, D), bias.reshape(1, D), maxes)


@jax.jit
def kernel(x, edge_index, edge_attr, W_l, b_l, W_r, b_r, W_e, att, bias):
    src = edge_index[0].astype(jnp.int32)
    dst = edge_index[1].astype(jnp.int32)
    pad = ((0, 0), (0, EPWP - EPW))
    srcw = jnp.pad(src.reshape(NW, EPW), pad)
    dstw = jnp.pad(dst.reshape(NW, EPW), pad)
    sdw = srcw * PK + dstw
    sd1 = sdw.reshape(NW, NB1, B1)
    sd = sdw.reshape(NW, NB2, B2)
    x_l, x_r, ew, ea_sum = _proj(x, W_l, b_l, W_r, b_r, edge_attr, W_e)
    alpha, maxes = _sc_alpha(x_l, x_r, ew, sd1, att)
    acc, asum = _sc_agg(x_l, sd, alpha, maxes)
    return _final(acc[0], acc[1], asum[0], asum[1], x_l, x_r,
                  ea_sum, W_e, att, bias, maxes)


# speedup vs baseline: 1.4320x; 1.4320x over previous
import jax
import jax.numpy as jnp
from jax import lax
from jax.experimental import pallas as pl
from jax.experimental.pallas import tpu as pltpu
from jax.experimental.pallas import tpu_sc as plsc

N = 10000
E = 320000
D = 128
DE = 16

NC = 2
NS = 16
NW = NC * NS
EPW = E // NW
B = 80
NB = EPW // B
RPT = N // NS
LRELU = 0.2
PK = 16384


def _proj_body(x_ref, wl_ref, bl_ref, wr_ref, br_ref, ea_ref, we_ref,
               xl_ref, xr_ref, ew_ref, easum_ref):
    xb = x_ref[...]
    xl_ref[...] = jnp.dot(xb, wl_ref[...], preferred_element_type=jnp.float32) + bl_ref[...]
    xr_ref[...] = jnp.dot(xb, wr_ref[...], preferred_element_type=jnp.float32) + br_ref[...]
    ea = ea_ref[...]
    ew_ref[...] = jnp.dot(ea, we_ref[...], preferred_element_type=jnp.float32)
    part = jnp.sum(ea, axis=0, keepdims=True)

    @pl.when(pl.program_id(0) == 0)
    def _():
        easum_ref[...] = part

    @pl.when(pl.program_id(0) != 0)
    def _():
        easum_ref[...] = easum_ref[...] + part


def _proj(x, W_l, b_l, W_r, b_r, edge_attr, W_e):
    grid = (25,)
    nblk = N // 25
    eblk = E // 25
    return pl.pallas_call(
        _proj_body,
        grid=grid,
        in_specs=[
            pl.BlockSpec((nblk, D), lambda i: (i, 0)),
            pl.BlockSpec((D, D), lambda i: (0, 0)),
            pl.BlockSpec((1, D), lambda i: (0, 0)),
            pl.BlockSpec((D, D), lambda i: (0, 0)),
            pl.BlockSpec((1, D), lambda i: (0, 0)),
            pl.BlockSpec((eblk, DE), lambda i: (i, 0)),
            pl.BlockSpec((DE, D), lambda i: (0, 0)),
        ],
        out_specs=[
            pl.BlockSpec((nblk, D), lambda i: (i, 0)),
            pl.BlockSpec((nblk, D), lambda i: (i, 0)),
            pl.BlockSpec((eblk, D), lambda i: (i, 0)),
            pl.BlockSpec((1, DE), lambda i: (0, 0)),
        ],
        out_shape=[
            jax.ShapeDtypeStruct((N, D), jnp.float32),
            jax.ShapeDtypeStruct((N, D), jnp.float32),
            jax.ShapeDtypeStruct((E, D), jnp.float32),
            jax.ShapeDtypeStruct((1, DE), jnp.float32),
        ],
    )(x, W_l, b_l.reshape(1, D), W_r, b_r.reshape(1, D), edge_attr, W_e)


def _sc_alpha_body(xl_hbm, xr_hbm, ew_hbm, src_hbm, dst_hbm, att_hbm,
                   alpha_hbm, maxes_hbm,
                   src_v, dst_v, xl_v, xr_v, ew_v, att_v, alpha_v, stage_v,
                   sem):
    cid = lax.axis_index("c")
    sid = lax.axis_index("s")
    wid = cid * NS + sid

    pltpu.sync_copy(src_hbm.at[wid], src_v)
    pltpu.sync_copy(dst_hbm.at[wid], dst_v)
    pltpu.sync_copy(att_hbm, att_v)

    lane_iota = lax.iota(jnp.int32, 16)

    def issue(b, p):
        pltpu.async_copy(xl_hbm.at[src_v.at[b]], xl_v.at[p], sem)
        pltpu.async_copy(xr_hbm.at[dst_v.at[b]], xr_v.at[p], sem)
        pltpu.async_copy(ew_hbm.at[pl.ds(wid * EPW + b * B, B)], ew_v.at[p],
                         sem)

    def drain(p):
        pltpu.make_async_copy(xl_hbm.at[pl.ds(0, B)], xl_v.at[p], sem).wait()
        pltpu.make_async_copy(xl_hbm.at[pl.ds(0, B)], xr_v.at[p], sem).wait()
        pltpu.make_async_copy(ew_hbm.at[pl.ds(0, B)], ew_v.at[p], sem).wait()

    issue(0, 0)
    issue(1, 1)

    def batch_body(b, _):
        p = lax.rem(b, 3)
        drain(p)

        @pl.when(b + 2 < NB)
        def _():
            issue(b + 2, lax.rem(b + 2, 3))

        @plsc.parallel_loop(0, B // 16, unroll=2)
        def group_body(g):
            alpha16 = jnp.zeros((16,), jnp.float32)
            for e in range(16):
                r = g * 16 + e
                acc0 = jnp.zeros((16,), jnp.float32)
                acc1 = jnp.zeros((16,), jnp.float32)
                for v in range(D // 16):
                    sl = pl.ds(v * 16, 16)
                    m = xl_v[p, r, sl] + xr_v[p, r, sl] + ew_v[p, r, sl]
                    m = jnp.maximum(m, LRELU * m)
                    if v % 2 == 0:
                        acc0 = acc0 + m * att_v[sl]
                    else:
                        acc1 = acc1 + m * att_v[sl]
                s = jnp.sum(acc0 + acc1)
                alpha16 = jnp.where(lane_iota == e, s, alpha16)
            alpha_v[pl.ds(b * B + g * 16, 16)] = alpha16

        return 0

    lax.fori_loop(0, NB, batch_body, 0)

    def max_body(i, m16):
        return jnp.maximum(m16, alpha_v[pl.ds(i * 16, 16)])

    m16 = lax.fori_loop(0, EPW // 16, max_body,
                        jnp.full((16,), -3.0e38, jnp.float32))
    pltpu.sync_copy(alpha_v, alpha_hbm.at[wid])
    stage_v[...] = jnp.full((16,), jnp.max(m16), jnp.float32)
    pltpu.sync_copy(stage_v, maxes_hbm.at[wid])


def _sc_alpha(x_l, x_r, ew, src, dst, att):
    mesh = plsc.VectorSubcoreMesh(core_axis_name="c", subcore_axis_name="s")
    f = pl.kernel(
        _sc_alpha_body,
        out_type=[
            jax.ShapeDtypeStruct((NW, EPW), jnp.float32),
            jax.ShapeDtypeStruct((NW, 16), jnp.float32),
        ],
        mesh=mesh,
        compiler_params=pltpu.CompilerParams(needs_layout_passes=False,
                                             use_tc_tiling_on_sc=False),
        scratch_types=[
            pltpu.VMEM((NB, B), jnp.int32),
            pltpu.VMEM((NB, B), jnp.int32),
            pltpu.VMEM((3, B, D), jnp.float32),
            pltpu.VMEM((3, B, D), jnp.float32),
            pltpu.VMEM((3, B, D), jnp.float32),
            pltpu.VMEM((D,), jnp.float32),
            pltpu.VMEM((EPW,), jnp.float32),
            pltpu.VMEM((16,), jnp.float32),
            pltpu.SemaphoreType.DMA,
        ],
    )
    return f(x_l, x_r, ew, src, dst, att)


def _sc_agg_body(xl_hbm, sd_hbm, alpha_hbm, maxes_hbm,
                 acc_hbm, asum_hbm,
                 sd_v, srcb_v, dstb_v, rows_v, aexp_v, alpha_b, maxes_v,
                 acc_sh, asum_sh, sem):
    cid = lax.axis_index("c")
    sid = lax.axis_index("s")
    wid = cid * NS + sid

    def zbuf_body(i, _):
        for v in range(D // 16):
            rows_v[0, i, pl.ds(v * 16, 16)] = jnp.zeros((16,), jnp.float32)
        aexp_v[i, :] = jnp.zeros((16,), jnp.float32)
        return 0

    lax.fori_loop(0, B, zbuf_body, 0)

    nchunk = N // 16

    def zspm_body(i, _):
        c = sid + NS * i

        @pl.when(c < nchunk)
        def _():
            pltpu.sync_copy(rows_v.at[0, pl.ds(0, 16)],
                            acc_sh.at[pl.ds(c * 16, 16)])
            pltpu.sync_copy(aexp_v.at[pl.ds(0, 16)],
                            asum_sh.at[pl.ds(c * 16, 16)])
        return 0

    lax.fori_loop(0, (nchunk + NS - 1) // NS, zspm_body, 0)
    plsc.subcore_barrier()

    pltpu.sync_copy(maxes_hbm, maxes_v)
    gv = maxes_v[0, :]
    for w in range(1, NW):
        gv = jnp.maximum(gv, maxes_v[w, :])
    gmax = jnp.max(gv)

    pltpu.sync_copy(sd_hbm.at[wid], sd_v)

    lane0 = (lax.iota(jnp.int32, 16) == 0).astype(jnp.float32)

    def unpack(b, p):
        for g0 in range(B // 16):
            sl = pl.ds(g0 * 16, 16)
            packed = sd_v[b, sl]
            srcb_v[p, sl] = lax.shift_right_logical(packed, 14)
            dstb_v[p, sl] = lax.bitwise_and(packed, PK - 1)

    def issue(b, p):
        pltpu.async_copy(xl_hbm.at[srcb_v.at[p]], rows_v.at[p], sem)
        pltpu.async_copy(alpha_hbm.at[wid, pl.ds(b * B, B)], alpha_b.at[p],
                         sem)

    def drain(p):
        pltpu.make_async_copy(xl_hbm.at[pl.ds(0, B)], rows_v.at[p], sem).wait()
        pltpu.make_async_copy(alpha_hbm.at[0, pl.ds(0, B)], alpha_b.at[p],
                              sem).wait()

    unpack(0, 0)
    issue(0, 0)

    def batch_body(b, _):
        p = lax.rem(b, 2)

        @pl.when(b + 1 < NB)
        def _():
            unpack(b + 1, 1 - p)

        drain(p)

        @pl.when(b + 1 < NB)
        def _():
            issue(b + 1, 1 - p)

        @plsc.parallel_loop(0, B // 16, unroll=2)
        def group_body(g):
            av = alpha_b[p, pl.ds(g * 16, 16)]
            ev16 = jnp.exp(av - gmax)
            for e in range(16):
                r = g * 16 + e
                evb = ev16.at[jnp.full((16,), e, jnp.int32)].get(
                    mode="promise_in_bounds")
                for v in range(D // 16):
                    sl = pl.ds(v * 16, 16)
                    rows_v[p, r, sl] = rows_v[p, r, sl] * evb
                aexp_v[r, :] = evb * lane0

        pltpu.sync_copy(rows_v.at[p], acc_sh.at[dstb_v.at[p]], add=True)
        pltpu.sync_copy(aexp_v, asum_sh.at[dstb_v.at[p]], add=True)
        return 0

    lax.fori_loop(0, NB, batch_body, 0)
    plsc.subcore_barrier()

    pltpu.sync_copy(acc_sh.at[pl.ds(sid * RPT, RPT)],
                    acc_hbm.at[cid, pl.ds(sid * RPT, RPT)])
    pltpu.sync_copy(asum_sh.at[pl.ds(sid * RPT, RPT)],
                    asum_hbm.at[cid, pl.ds(sid * RPT, RPT)])


def _sc_agg(x_l, sd, alpha, maxes):
    mesh = plsc.VectorSubcoreMesh(core_axis_name="c", subcore_axis_name="s")
    f = pl.kernel(
        _sc_agg_body,
        out_type=[
            jax.ShapeDtypeStruct((NC, N, D), jnp.float32),
            jax.ShapeDtypeStruct((NC, N, 16), jnp.float32),
        ],
        mesh=mesh,
        compiler_params=pltpu.CompilerParams(needs_layout_passes=False,
                                             use_tc_tiling_on_sc=False),
        scratch_types=[
            pltpu.VMEM((NB, B), jnp.int32),
            pltpu.VMEM((2, B), jnp.int32),
            pltpu.VMEM((2, B), jnp.int32),
            pltpu.VMEM((2, B, D), jnp.float32),
            pltpu.VMEM((B, 16), jnp.float32),
            pltpu.VMEM((2, B), jnp.float32),
            pltpu.VMEM((NW, 16), jnp.float32),
            pltpu.VMEM_SHARED((N, D), jnp.float32),
            pltpu.VMEM_SHARED((N, 16), jnp.float32),
            pltpu.SemaphoreType.DMA,
        ],
    )
    return f(x_l, sd, alpha, maxes)


def _final_body(acc0_ref, acc1_ref, as0_ref, as1_ref, xl_ref, xr_ref,
                easum_ref, we_ref, att_ref, bias_ref, maxes_ref, out_ref):
    gmax = jnp.max(maxes_ref[...])
    ea_mean = easum_ref[...] * (1.0 / E)
    c = jnp.sum(we_ref[...] * ea_mean.reshape(DE, 1), axis=0, keepdims=True)
    xl = xl_ref[...]
    m_self = xl + xr_ref[...] + c
    m_self = jnp.maximum(m_self, LRELU * m_self)
    alpha_self = jnp.sum(m_self * att_ref[...], axis=1, keepdims=True)
    self_aexp = jnp.exp(alpha_self - gmax)
    asum = (jnp.sum(as0_ref[...], axis=1, keepdims=True)
            + jnp.sum(as1_ref[...], axis=1, keepdims=True) + self_aexp)
    numer = acc0_ref[...] + acc1_ref[...] + self_aexp * xl
    h = numer / (asum + 1e-16) + bias_ref[...]
    out_ref[...] = jnp.maximum(h, 0.01 * h)


def _final(acc0, acc1, asum0, asum1, x_l, x_r, ea_sum, W_e, att, bias, maxes):
    blk = 1000
    grid = (N // blk,)
    return pl.pallas_call(
        _final_body,
        grid=grid,
        in_specs=[
            pl.BlockSpec((blk, D), lambda i: (i, 0)),
            pl.BlockSpec((blk, D), lambda i: (i, 0)),
            pl.BlockSpec((blk, 16), lambda i: (i, 0)),
            pl.BlockSpec((blk, 16), lambda i: (i, 0)),
            pl.BlockSpec((blk, D), lambda i: (i, 0)),
            pl.BlockSpec((blk, D), lambda i: (i, 0)),
            pl.BlockSpec((1, DE), lambda i: (0, 0)),
            pl.BlockSpec((DE, D), lambda i: (0, 0)),
            pl.BlockSpec((1, D), lambda i: (0, 0)),
            pl.BlockSpec((1, D), lambda i: (0, 0)),
            pl.BlockSpec((NW, 16), lambda i: (0, 0)),
        ],
        out_specs=pl.BlockSpec((blk, D), lambda i: (i, 0)),
        out_shape=jax.ShapeDtypeStruct((N, D), jnp.float32),
    )(acc0, acc1, asum0, asum1, x_l, x_r, ea_sum, W_e,
      att.reshape(1, D), bias.reshape(1, D), maxes)


@jax.jit
def kernel(x, edge_index, edge_attr, W_l, b_l, W_r, b_r, W_e, att, bias):
    src = edge_index[0].astype(jnp.int32)
    dst = edge_index[1].astype(jnp.int32)
    src3 = src.reshape(NW, NB, B)
    dst3 = dst.reshape(NW, NB, B)
    sd = (src * PK + dst).reshape(NW, NB, B)
    x_l, x_r, ew, ea_sum = _proj(x, W_l, b_l, W_r, b_r, edge_attr, W_e)
    alpha, maxes = _sc_alpha(x_l, x_r, ew, src3, dst3, att)
    acc, asum = _sc_agg(x_l, sd, alpha, maxes)
    return _final(acc[0], acc[1], asum[0], asum[1], x_l, x_r,
                  ea_sum, W_e, att, bias, maxes)


# speedup vs baseline: 1.4435x; 1.0080x over previous
import jax
import jax.numpy as jnp
from jax import lax
from jax.experimental import pallas as pl
from jax.experimental.pallas import tpu as pltpu
from jax.experimental.pallas import tpu_sc as plsc

N = 10000
E = 320000
D = 128
DE = 16

NC = 2
NS = 16
NW = NC * NS
EPW = E // NW
B = 80
NB = EPW // B
RPT = N // NS
LRELU = 0.2
PK = 16384


def _proj_body(x_ref, wl_ref, bl_ref, wr_ref, br_ref, ea_ref, we_ref,
               xl_ref, xr_ref, ew_ref, easum_ref):
    xb = x_ref[...]
    xl_ref[...] = jnp.dot(xb, wl_ref[...], preferred_element_type=jnp.float32) + bl_ref[...]
    xr_ref[...] = jnp.dot(xb, wr_ref[...], preferred_element_type=jnp.float32) + br_ref[...]
    ea = ea_ref[...]
    ew_ref[...] = jnp.dot(ea, we_ref[...], preferred_element_type=jnp.float32)
    part = jnp.sum(ea, axis=0, keepdims=True)

    @pl.when(pl.program_id(0) == 0)
    def _():
        easum_ref[...] = part

    @pl.when(pl.program_id(0) != 0)
    def _():
        easum_ref[...] = easum_ref[...] + part


def _proj(x, W_l, b_l, W_r, b_r, edge_attr, W_e):
    grid = (25,)
    nblk = N // 25
    eblk = E // 25
    return pl.pallas_call(
        _proj_body,
        grid=grid,
        in_specs=[
            pl.BlockSpec((nblk, D), lambda i: (i, 0)),
            pl.BlockSpec((D, D), lambda i: (0, 0)),
            pl.BlockSpec((1, D), lambda i: (0, 0)),
            pl.BlockSpec((D, D), lambda i: (0, 0)),
            pl.BlockSpec((1, D), lambda i: (0, 0)),
            pl.BlockSpec((eblk, DE), lambda i: (i, 0)),
            pl.BlockSpec((DE, D), lambda i: (0, 0)),
        ],
        out_specs=[
            pl.BlockSpec((nblk, D), lambda i: (i, 0)),
            pl.BlockSpec((nblk, D), lambda i: (i, 0)),
            pl.BlockSpec((eblk, D), lambda i: (i, 0)),
            pl.BlockSpec((1, DE), lambda i: (0, 0)),
        ],
        out_shape=[
            jax.ShapeDtypeStruct((N, D), jnp.float32),
            jax.ShapeDtypeStruct((N, D), jnp.float32),
            jax.ShapeDtypeStruct((E, D), jnp.float32),
            jax.ShapeDtypeStruct((1, DE), jnp.float32),
        ],
    )(x, W_l, b_l.reshape(1, D), W_r, b_r.reshape(1, D), edge_attr, W_e)


def _sc_alpha_body(xl_hbm, xr_hbm, ew_hbm, src_hbm, dst_hbm, att_hbm,
                   alpha_hbm, maxes_hbm,
                   src_v, dst_v, xl_v, xr_v, ew_v, att_v, alpha_v, stage_v,
                   sem):
    cid = lax.axis_index("c")
    sid = lax.axis_index("s")
    wid = cid * NS + sid

    pltpu.sync_copy(src_hbm.at[wid], src_v)
    pltpu.sync_copy(dst_hbm.at[wid], dst_v)
    pltpu.sync_copy(att_hbm, att_v)

    lane_iota = lax.iota(jnp.int32, 16)

    def issue(b, p):
        pltpu.async_copy(xl_hbm.at[src_v.at[b]], xl_v.at[p], sem)
        pltpu.async_copy(xr_hbm.at[dst_v.at[b]], xr_v.at[p], sem)
        pltpu.async_copy(ew_hbm.at[pl.ds(wid * EPW + b * B, B)], ew_v.at[p],
                         sem)

    def drain(p):
        pltpu.make_async_copy(xl_hbm.at[pl.ds(0, B)], xl_v.at[p], sem).wait()
        pltpu.make_async_copy(xl_hbm.at[pl.ds(0, B)], xr_v.at[p], sem).wait()
        pltpu.make_async_copy(ew_hbm.at[pl.ds(0, B)], ew_v.at[p], sem).wait()

    issue(0, 0)
    issue(1, 1)

    def batch_body(b, _):
        p = lax.rem(b, 3)
        drain(p)

        @pl.when(b + 2 < NB)
        def _():
            issue(b + 2, lax.rem(b + 2, 3))

        @plsc.parallel_loop(0, B // 16, unroll=2)
        def group_body(g):
            alpha16 = jnp.zeros((16,), jnp.float32)
            for e in range(16):
                r = g * 16 + e
                acc0 = jnp.zeros((16,), jnp.float32)
                acc1 = jnp.zeros((16,), jnp.float32)
                for v in range(D // 16):
                    sl = pl.ds(v * 16, 16)
                    m = xl_v[p, r, sl] + xr_v[p, r, sl] + ew_v[p, r, sl]
                    m = jnp.maximum(m, LRELU * m)
                    if v % 2 == 0:
                        acc0 = acc0 + m * att_v[sl]
                    else:
                        acc1 = acc1 + m * att_v[sl]
                s = jnp.sum(acc0 + acc1)
                alpha16 = jnp.where(lane_iota == e, s, alpha16)
            alpha_v[pl.ds(b * B + g * 16, 16)] = alpha16

        return 0

    lax.fori_loop(0, NB, batch_body, 0)

    def max_body(i, m16):
        return jnp.maximum(m16, alpha_v[pl.ds(i * 16, 16)])

    m16 = lax.fori_loop(0, EPW // 16, max_body,
                        jnp.full((16,), -3.0e38, jnp.float32))
    pltpu.sync_copy(alpha_v, alpha_hbm.at[wid])
    stage_v[...] = jnp.full((16,), jnp.max(m16), jnp.float32)
    pltpu.sync_copy(stage_v, maxes_hbm.at[wid])


def _sc_alpha(x_l, x_r, ew, src, dst, att):
    mesh = plsc.VectorSubcoreMesh(core_axis_name="c", subcore_axis_name="s")
    f = pl.kernel(
        _sc_alpha_body,
        out_type=[
            jax.ShapeDtypeStruct((NW, EPW), jnp.float32),
            jax.ShapeDtypeStruct((NW, 16), jnp.float32),
        ],
        mesh=mesh,
        compiler_params=pltpu.CompilerParams(needs_layout_passes=False,
                                             use_tc_tiling_on_sc=False),
        scratch_types=[
            pltpu.VMEM((NB, B), jnp.int32),
            pltpu.VMEM((NB, B), jnp.int32),
            pltpu.VMEM((3, B, D), jnp.float32),
            pltpu.VMEM((3, B, D), jnp.float32),
            pltpu.VMEM((3, B, D), jnp.float32),
            pltpu.VMEM((D,), jnp.float32),
            pltpu.VMEM((EPW,), jnp.float32),
            pltpu.VMEM((16,), jnp.float32),
            pltpu.SemaphoreType.DMA,
        ],
    )
    return f(x_l, x_r, ew, src, dst, att)


def _sc_agg_body(xl_hbm, sd_hbm, alpha_hbm, maxes_hbm,
                 acc_hbm, asum_hbm,
                 sd_v, srcb_v, dstb_v, rows_v, aexp_v, alpha_b, maxes_v,
                 acc_sh, asum_sh, sem, sem2):
    cid = lax.axis_index("c")
    sid = lax.axis_index("s")
    wid = cid * NS + sid

    def zbuf_body(i, _):
        for v in range(D // 16):
            rows_v[0, i, pl.ds(v * 16, 16)] = jnp.zeros((16,), jnp.float32)
        aexp_v[0, i, :] = jnp.zeros((16,), jnp.float32)
        return 0

    lax.fori_loop(0, B, zbuf_body, 0)

    nchunk = N // 16

    def zspm_body(i, _):
        c = sid + NS * i

        @pl.when(c < nchunk)
        def _():
            pltpu.sync_copy(rows_v.at[0, pl.ds(0, 16)],
                            acc_sh.at[pl.ds(c * 16, 16)])
            pltpu.sync_copy(aexp_v.at[0, pl.ds(0, 16)],
                            asum_sh.at[pl.ds(c * 16, 16)])
        return 0

    lax.fori_loop(0, (nchunk + NS - 1) // NS, zspm_body, 0)
    plsc.subcore_barrier()

    pltpu.sync_copy(maxes_hbm, maxes_v)
    gv = maxes_v[0, :]
    for w in range(1, NW):
        gv = jnp.maximum(gv, maxes_v[w, :])
    gmax = jnp.max(gv)

    pltpu.sync_copy(sd_hbm.at[wid], sd_v)

    lane0 = (lax.iota(jnp.int32, 16) == 0).astype(jnp.float32)

    def unpack(b, p):
        for g0 in range(B // 16):
            sl = pl.ds(g0 * 16, 16)
            packed = sd_v[b, sl]
            srcb_v[p, sl] = lax.shift_right_logical(packed, 14)
            dstb_v[p, sl] = lax.bitwise_and(packed, PK - 1)

    def issue(b, p):
        pltpu.async_copy(xl_hbm.at[srcb_v.at[p]], rows_v.at[p], sem)
        pltpu.async_copy(alpha_hbm.at[wid, pl.ds(b * B, B)], alpha_b.at[p],
                         sem)

    def drain(p):
        pltpu.make_async_copy(xl_hbm.at[pl.ds(0, B)], rows_v.at[p], sem).wait()
        pltpu.make_async_copy(alpha_hbm.at[0, pl.ds(0, B)], alpha_b.at[p],
                              sem).wait()

    unpack(0, 0)
    issue(0, 0)

    def drain_scatter(p):
        pltpu.make_async_copy(rows_v.at[p], acc_sh.at[dstb_v.at[p]],
                              sem2).wait()
        pltpu.make_async_copy(aexp_v.at[p], asum_sh.at[dstb_v.at[p]],
                              sem2).wait()

    def batch_body(b, _):
        p = lax.rem(b, 2)

        @pl.when(b >= 1)
        def _():
            drain_scatter(1 - p)

        @pl.when(b + 1 < NB)
        def _():
            unpack(b + 1, 1 - p)

        drain(p)

        @pl.when(b + 1 < NB)
        def _():
            issue(b + 1, 1 - p)

        @plsc.parallel_loop(0, B // 16, unroll=2)
        def group_body(g):
            av = alpha_b[p, pl.ds(g * 16, 16)]
            ev16 = jnp.exp(av - gmax)
            for e in range(16):
                r = g * 16 + e
                evb = ev16.at[jnp.full((16,), e, jnp.int32)].get(
                    mode="promise_in_bounds")
                for v in range(D // 16):
                    sl = pl.ds(v * 16, 16)
                    rows_v[p, r, sl] = rows_v[p, r, sl] * evb
                aexp_v[p, r, :] = evb * lane0

        pltpu.async_copy(rows_v.at[p], acc_sh.at[dstb_v.at[p]], sem2,
                         add=True)
        pltpu.async_copy(aexp_v.at[p], asum_sh.at[dstb_v.at[p]], sem2,
                         add=True)
        return 0

    lax.fori_loop(0, NB, batch_body, 0)
    drain_scatter((NB - 1) % 2)
    plsc.subcore_barrier()

    pltpu.sync_copy(acc_sh.at[pl.ds(sid * RPT, RPT)],
                    acc_hbm.at[cid, pl.ds(sid * RPT, RPT)])
    pltpu.sync_copy(asum_sh.at[pl.ds(sid * RPT, RPT)],
                    asum_hbm.at[cid, pl.ds(sid * RPT, RPT)])


def _sc_agg(x_l, sd, alpha, maxes):
    mesh = plsc.VectorSubcoreMesh(core_axis_name="c", subcore_axis_name="s")
    f = pl.kernel(
        _sc_agg_body,
        out_type=[
            jax.ShapeDtypeStruct((NC, N, D), jnp.float32),
            jax.ShapeDtypeStruct((NC, N, 16), jnp.float32),
        ],
        mesh=mesh,
        compiler_params=pltpu.CompilerParams(needs_layout_passes=False,
                                             use_tc_tiling_on_sc=False),
        scratch_types=[
            pltpu.VMEM((NB, B), jnp.int32),
            pltpu.VMEM((2, B), jnp.int32),
            pltpu.VMEM((2, B), jnp.int32),
            pltpu.VMEM((2, B, D), jnp.float32),
            pltpu.VMEM((2, B, 16), jnp.float32),
            pltpu.VMEM((2, B), jnp.float32),
            pltpu.VMEM((NW, 16), jnp.float32),
            pltpu.VMEM_SHARED((N, D), jnp.float32),
            pltpu.VMEM_SHARED((N, 16), jnp.float32),
            pltpu.SemaphoreType.DMA,
            pltpu.SemaphoreType.DMA,
        ],
    )
    return f(x_l, sd, alpha, maxes)


def _final_body(acc0_ref, acc1_ref, as0_ref, as1_ref, xl_ref, xr_ref,
                easum_ref, we_ref, att_ref, bias_ref, maxes_ref, out_ref):
    gmax = jnp.max(maxes_ref[...])
    ea_mean = easum_ref[...] * (1.0 / E)
    c = jnp.sum(we_ref[...] * ea_mean.reshape(DE, 1), axis=0, keepdims=True)
    xl = xl_ref[...]
    m_self = xl + xr_ref[...] + c
    m_self = jnp.maximum(m_self, LRELU * m_self)
    alpha_self = jnp.sum(m_self * att_ref[...], axis=1, keepdims=True)
    self_aexp = jnp.exp(alpha_self - gmax)
    asum = (jnp.sum(as0_ref[...], axis=1, keepdims=True)
            + jnp.sum(as1_ref[...], axis=1, keepdims=True) + self_aexp)
    numer = acc0_ref[...] + acc1_ref[...] + self_aexp * xl
    h = numer / (asum + 1e-16) + bias_ref[...]
    out_ref[...] = jnp.maximum(h, 0.01 * h)


def _final(acc0, acc1, asum0, asum1, x_l, x_r, ea_sum, W_e, att, bias, maxes):
    blk = 1000
    grid = (N // blk,)
    return pl.pallas_call(
        _final_body,
        grid=grid,
        in_specs=[
            pl.BlockSpec((blk, D), lambda i: (i, 0)),
            pl.BlockSpec((blk, D), lambda i: (i, 0)),
            pl.BlockSpec((blk, 16), lambda i: (i, 0)),
            pl.BlockSpec((blk, 16), lambda i: (i, 0)),
            pl.BlockSpec((blk, D), lambda i: (i, 0)),
            pl.BlockSpec((blk, D), lambda i: (i, 0)),
            pl.BlockSpec((1, DE), lambda i: (0, 0)),
            pl.BlockSpec((DE, D), lambda i: (0, 0)),
            pl.BlockSpec((1, D), lambda i: (0, 0)),
            pl.BlockSpec((1, D), lambda i: (0, 0)),
            pl.BlockSpec((NW, 16), lambda i: (0, 0)),
        ],
        out_specs=pl.BlockSpec((blk, D), lambda i: (i, 0)),
        out_shape=jax.ShapeDtypeStruct((N, D), jnp.float32),
    )(acc0, acc1, asum0, asum1, x_l, x_r, ea_sum, W_e,
      att.reshape(1, D), bias.reshape(1, D), maxes)


@jax.jit
def kernel(x, edge_index, edge_attr, W_l, b_l, W_r, b_r, W_e, att, bias):
    src = edge_index[0].astype(jnp.int32)
    dst = edge_index[1].astype(jnp.int32)
    src3 = src.reshape(NW, NB, B)
    dst3 = dst.reshape(NW, NB, B)
    sd = (src * PK + dst).reshape(NW, NB, B)
    x_l, x_r, ew, ea_sum = _proj(x, W_l, b_l, W_r, b_r, edge_attr, W_e)
    alpha, maxes = _sc_alpha(x_l, x_r, ew, src3, dst3, att)
    acc, asum = _sc_agg(x_l, sd, alpha, maxes)
    return _final(acc[0], acc[1], asum[0], asum[1], x_l, x_r,
                  ea_sum, W_e, att, bias, maxes)
